# Initial kernel scaffold; baseline (speedup 1.0000x reference)
import jax
import jax.numpy as jnp
from jax.experimental import pallas as pl


def kernel(weight, indices):
    raise NotImplementedError("write your pallas kernel here")


# speedup vs baseline: 1.1635x; 1.1635x over previous
import functools
import jax
import jax.numpy as jnp
from jax import lax
from jax.experimental import pallas as pl
from jax.experimental.pallas import tpu as pltpu
from jax.experimental.pallas import tpu_sc as plsc

_NC = 2
_NS = 16
_NW = _NC * _NS
_CHUNK = 128


@functools.partial(jax.jit, static_argnums=())
def _gather(weight, idx3):
    _, per_w_rows, _ = idx3.shape
    B = _NW * per_w_rows * _CHUNK
    D = weight.shape[1]
    per_w = per_w_rows * _CHUNK

    mesh = plsc.VectorSubcoreMesh(core_axis_name="c", subcore_axis_name="s")

    @functools.partial(
        pl.kernel,
        mesh=mesh,
        out_type=jax.ShapeDtypeStruct((B, D), jnp.float32),
        scratch_types=[
            pltpu.VMEM((per_w_rows, _CHUNK), jnp.int32),
            pltpu.VMEM((_CHUNK, D), jnp.float32),
            pltpu.SemaphoreType.DMA,
        ],
    )
    def k(idx_hbm, table_hbm, out_hbm, idx_v, rows_v, sem):
        wid = lax.axis_index("s") * _NC + lax.axis_index("c")
        base = wid * per_w
        pltpu.sync_copy(idx_hbm.at[wid], idx_v)

        def step(j, carry):
            pltpu.async_copy(table_hbm.at[idx_v.at[j]], rows_v, sem).wait()
            pltpu.sync_copy(rows_v, out_hbm.at[pl.ds(base + j * _CHUNK, _CHUNK)])
            return carry

        lax.fori_loop(0, per_w_rows, step, 0)

    return k(idx3, weight)


def kernel(weight, indices):
    b, f = indices.shape
    d = weight.shape[1]
    idx = indices.reshape(-1).astype(jnp.int32)
    idx3 = idx.reshape(_NW, -1, _CHUNK)
    out = _gather(weight, idx3)
    return out.reshape(b, f, d)


# speedup vs baseline: 1.2532x; 1.0771x over previous
import functools
import jax
import jax.numpy as jnp
from jax import lax
from jax.experimental import pallas as pl
from jax.experimental.pallas import tpu as pltpu
from jax.experimental.pallas import tpu_sc as plsc

_NC = 2
_NS = 16
_NW = _NC * _NS
_CHUNK = 128


@functools.partial(jax.jit, static_argnums=())
def _gather(weight, idx3):
    _, per_w_rows, _ = idx3.shape
    B = _NW * per_w_rows * _CHUNK
    D = weight.shape[1]
    per_w = per_w_rows * _CHUNK

    mesh = plsc.VectorSubcoreMesh(core_axis_name="c", subcore_axis_name="s")

    n_groups = per_w_rows // 2

    @functools.partial(
        pl.kernel,
        mesh=mesh,
        out_type=jax.ShapeDtypeStruct((B, D), jnp.float32),
        scratch_types=[
            pltpu.VMEM((per_w_rows, _CHUNK), jnp.int32),
            pltpu.VMEM((_CHUNK, D), jnp.float32),
            pltpu.VMEM((_CHUNK, D), jnp.float32),
            pltpu.SemaphoreType.DMA,
            pltpu.SemaphoreType.DMA,
            pltpu.SemaphoreType.DMA,
            pltpu.SemaphoreType.DMA,
        ],
    )
    def k(idx_hbm, table_hbm, out_hbm, idx_v, rows_a, rows_b, ga, gb, wa, wb):
        wid = lax.axis_index("s") * _NC + lax.axis_index("c")
        base = wid * per_w
        pltpu.sync_copy(idx_hbm.at[wid], idx_v)

        def g_start(j, buf, sem):
            pltpu.async_copy(table_hbm.at[idx_v.at[j]], buf, sem)

        def g_wait(buf, sem):
            pltpu.make_async_copy(table_hbm.at[idx_v.at[0]], buf, sem).wait()

        def w_start(j, buf, sem):
            pltpu.async_copy(buf, out_hbm.at[pl.ds(base + j * _CHUNK, _CHUNK)], sem)

        def w_wait(buf, sem):
            pltpu.make_async_copy(buf, out_hbm.at[pl.ds(base, _CHUNK)], sem).wait()

        g_start(0, rows_a, ga)
        g_start(1, rows_b, gb)

        def body(g, carry):
            j0 = 2 * g
            g_wait(rows_a, ga)
            w_start(j0, rows_a, wa)
            g_wait(rows_b, gb)
            w_start(j0 + 1, rows_b, wb)

            @pl.when(g < n_groups - 1)
            def _():
                w_wait(rows_a, wa)
                g_start(j0 + 2, rows_a, ga)
                w_wait(rows_b, wb)
                g_start(j0 + 3, rows_b, gb)

            return carry

        lax.fori_loop(0, n_groups, body, 0)
        w_wait(rows_a, wa)
        w_wait(rows_b, wb)

    return k(idx3, weight)


def kernel(weight, indices):
    b, f = indices.shape
    d = weight.shape[1]
    idx = indices.reshape(-1).astype(jnp.int32)
    idx3 = idx.reshape(_NW, -1, _CHUNK)
    out = _gather(weight, idx3)
    return out.reshape(b, f, d)


# speedup vs baseline: 1.2775x; 1.0193x over previous
import functools
import jax
import jax.numpy as jnp
from jax import lax
from jax.experimental import pallas as pl
from jax.experimental.pallas import tpu as pltpu
from jax.experimental.pallas import tpu_sc as plsc

_NC = 2
_NS = 16
_NW = _NC * _NS
_CHUNK = 128


@functools.partial(jax.jit, static_argnums=())
def _gather(weight, idx3):
    _, per_w_rows, _ = idx3.shape
    B = _NW * per_w_rows * _CHUNK
    D = weight.shape[1]
    per_w = per_w_rows * _CHUNK

    mesh = plsc.VectorSubcoreMesh(core_axis_name="c", subcore_axis_name="s")

    NBUF = 4
    n_main = per_w_rows // NBUF
    n_tail = per_w_rows - n_main * NBUF

    @functools.partial(
        pl.kernel,
        mesh=mesh,
        out_type=jax.ShapeDtypeStruct((B, D), jnp.float32),
        scratch_types=[
            pltpu.VMEM((per_w_rows, _CHUNK), jnp.int32),
        ]
        + [pltpu.VMEM((_CHUNK, D), jnp.float32)] * NBUF
        + [pltpu.SemaphoreType.DMA] * (2 * NBUF),
    )
    def k(idx_hbm, table_hbm, out_hbm, idx_v, *bufs_sems):
        bufs = bufs_sems[:NBUF]
        gsem = bufs_sems[NBUF : 2 * NBUF]
        wsem = bufs_sems[2 * NBUF :]
        wid = lax.axis_index("s") * _NC + lax.axis_index("c")
        base = wid * per_w
        pltpu.sync_copy(idx_hbm.at[wid], idx_v)

        def g_start(j, b):
            pltpu.async_copy(table_hbm.at[idx_v.at[j]], bufs[b], gsem[b])

        def g_wait(b):
            pltpu.make_async_copy(table_hbm.at[idx_v.at[0]], bufs[b], gsem[b]).wait()

        def w_start(j, b):
            pltpu.async_copy(bufs[b], out_hbm.at[pl.ds(base + j * _CHUNK, _CHUNK)], wsem[b])

        def w_wait(b):
            pltpu.make_async_copy(bufs[b], out_hbm.at[pl.ds(base, _CHUNK)], wsem[b]).wait()

        for b in range(NBUF):
            g_start(b, b)

        def body(g, carry):
            j0 = g * NBUF
            for b in range(NBUF):
                g_wait(b)
                w_start(j0 + b, b)
            for b in range(NBUF):
                nj = j0 + b + NBUF

                @pl.when(nj < per_w_rows)
                def _(nj=nj, b=b):
                    w_wait(b)
                    g_start(nj, b)

            return carry

        lax.fori_loop(0, n_main, body, 0)
        for b in range(n_tail):
            g_wait(b)
            w_start(n_main * NBUF + b, b)
        for b in range(NBUF):
            w_wait(b)

    return k(idx3, weight)


def kernel(weight, indices):
    b, f = indices.shape
    d = weight.shape[1]
    idx = indices.reshape(-1).astype(jnp.int32)
    idx3 = idx.reshape(_NW, -1, _CHUNK)
    out = _gather(weight, idx3)
    return out.reshape(b, f, d)


# speedup vs baseline: 2.0101x; 1.5735x over previous
import functools
import jax
import jax.numpy as jnp
from jax import lax
from jax.experimental import pallas as pl
from jax.experimental.pallas import tpu as pltpu
from jax.experimental.pallas import tpu_sc as plsc

_NC = 2
_NS = 16
_NW = _NC * _NS
_PG = 4


def _gather(weight, idx3, b, f):
    _, n_chunks, chunk_idx = idx3.shape
    D = weight.shape[1]
    planes_per_w = b // _NW
    NBUF = 4
    n_main = n_chunks // NBUF

    mesh = plsc.VectorSubcoreMesh(core_axis_name="c", subcore_axis_name="s")

    @functools.partial(
        pl.kernel,
        mesh=mesh,
        out_type=jax.ShapeDtypeStruct((b, f, D), jnp.float32),
        scratch_types=[
            pltpu.VMEM((n_chunks, chunk_idx), jnp.int32),
        ]
        + [pltpu.VMEM((_PG * f, D), jnp.float32)] * NBUF
        + [pltpu.SemaphoreType.DMA] * (2 * NBUF),
    )
    def k(idx_hbm, table_hbm, out_hbm, idx_v, *bufs_sems):
        bufs = bufs_sems[:NBUF]
        gsem = bufs_sems[NBUF : 2 * NBUF]
        wsem = bufs_sems[2 * NBUF :]
        wid = lax.axis_index("s") * _NC + lax.axis_index("c")
        plane0 = wid * planes_per_w
        pltpu.sync_copy(idx_hbm.at[wid], idx_v)

        def g_start(c, bf):
            pltpu.async_copy(table_hbm.at[idx_v.at[c]], bufs[bf], gsem[bf])

        def g_wait(bf):
            pltpu.make_async_copy(table_hbm.at[idx_v.at[0]], bufs[bf], gsem[bf]).wait()

        def w_start(c, bf):
            for p in range(_PG):
                pltpu.async_copy(
                    bufs[bf].at[pl.ds(p * f, f)],
                    out_hbm.at[plane0 + c * _PG + p],
                    wsem[bf],
                )

        def w_wait(bf):
            for p in range(_PG):
                pltpu.make_async_copy(
                    bufs[bf].at[pl.ds(p * f, f)], out_hbm.at[plane0], wsem[bf]
                ).wait()

        for bf in range(NBUF):
            g_start(bf, bf)

        def body(g, carry):
            c0 = g * NBUF
            for bf in range(NBUF):
                g_wait(bf)
                w_start(c0 + bf, bf)
            for bf in range(NBUF):
                nc = c0 + bf + NBUF

                @pl.when(nc < n_chunks)
                def _(nc=nc, bf=bf):
                    w_wait(bf)
                    g_start(nc, bf)

            return carry

        lax.fori_loop(0, n_main, body, 0)
        for bf in range(n_chunks - n_main * NBUF):
            g_wait(bf)
            w_start(n_main * NBUF + bf, bf)
        for bf in range(NBUF):
            w_wait(bf)

    return k(idx3, weight)


def kernel(weight, indices):
    b, f = indices.shape
    d = weight.shape[1]
    idx = indices.reshape(-1).astype(jnp.int32)
    idx3 = idx.reshape(_NW, -1, _PG * f)
    return _gather(weight, idx3, b, f)


# speedup vs baseline: 2.0213x; 1.0056x over previous
import functools
import jax
import jax.numpy as jnp
from jax import lax
from jax.experimental import pallas as pl
from jax.experimental.pallas import tpu as pltpu
from jax.experimental.pallas import tpu_sc as plsc

_NC = 2
_NS = 16
_NW = _NC * _NS
_PG = 4


def _gather(weight, idx3, b, f):
    _, n_chunks, chunk_idx = idx3.shape
    D = weight.shape[1]
    planes_per_w = b // _NW
    NBUF = 4
    n_main = n_chunks // NBUF

    mesh = plsc.VectorSubcoreMesh(core_axis_name="c", subcore_axis_name="s")

    @functools.partial(
        pl.kernel,
        mesh=mesh,
        compiler_params=pltpu.CompilerParams(use_tc_tiling_on_sc=True),
        out_type=jax.ShapeDtypeStruct((b, f, D), jnp.float32),
        scratch_types=[
            pltpu.VMEM((n_chunks, chunk_idx), jnp.int32),
        ]
        + [pltpu.VMEM((_PG * f, D), jnp.float32)] * NBUF
        + [pltpu.SemaphoreType.DMA] * (2 * NBUF),
    )
    def k(idx_hbm, table_hbm, out_hbm, idx_v, *bufs_sems):
        bufs = bufs_sems[:NBUF]
        gsem = bufs_sems[NBUF : 2 * NBUF]
        wsem = bufs_sems[2 * NBUF :]
        wid = lax.axis_index("s") * _NC + lax.axis_index("c")
        plane0 = wid * planes_per_w
        pltpu.sync_copy(idx_hbm.at[wid], idx_v)

        def g_start(c, bf):
            pltpu.async_copy(table_hbm.at[idx_v.at[c]], bufs[bf], gsem[bf])

        def g_wait(bf):
            pltpu.make_async_copy(table_hbm.at[idx_v.at[0]], bufs[bf], gsem[bf]).wait()

        def w_start(c, bf):
            for p in range(_PG):
                pltpu.async_copy(
                    bufs[bf].at[pl.ds(p * f, f)],
                    out_hbm.at[plane0 + c * _PG + p],
                    wsem[bf],
                )

        def w_wait(bf):
            for p in range(_PG):
                pltpu.make_async_copy(
                    bufs[bf].at[pl.ds(p * f, f)], out_hbm.at[plane0], wsem[bf]
                ).wait()

        for bf in range(NBUF):
            g_start(bf, bf)

        def body(g, carry):
            c0 = g * NBUF
            for bf in range(NBUF):
                g_wait(bf)
                w_start(c0 + bf, bf)
            for bf in range(NBUF):
                nc = c0 + bf + NBUF

                @pl.when(nc < n_chunks)
                def _(nc=nc, bf=bf):
                    w_wait(bf)
                    g_start(nc, bf)

            return carry

        lax.fori_loop(0, n_main, body, 0)
        for bf in range(n_chunks - n_main * NBUF):
            g_wait(bf)
            w_start(n_main * NBUF + bf, bf)
        for bf in range(NBUF):
            w_wait(bf)

    return k(idx3, weight)


def kernel(weight, indices):
    b, f = indices.shape
    d = weight.shape[1]
    idx = indices.reshape(-1).astype(jnp.int32)
    idx3 = idx.reshape(_NW, -1, _PG * f)
    return _gather(weight, idx3, b, f)
